# baseline (speedup 1.0000x reference)
import functools

import jax
import jax.numpy as jnp
from jax import lax
from jax.experimental import pallas as pl
from jax.experimental.pallas import tpu as pltpu
from jax.experimental.pallas import tpu_sc as plsc

N_ROWS = 1000000
D = 32
B = 16384
L = 16
NW = 32


BC = 100352
GRID = (N_ROWS + BC - 1) // BC


def _matvec_body(utt_ref, w_ref, b_ref, yu_ref):
    u = utt_ref[...]
    wu = w_ref[0:1, :D]
    yu = jax.lax.dot_general(wu, u, (((1,), (0,)), ((), ())),
                             preferred_element_type=jnp.float32)
    yu_ref[...] = yu[0] + b_ref[0]


_matvec = pl.pallas_call(
    _matvec_body,
    grid=(GRID,),
    in_specs=[
        pl.BlockSpec((D, BC), lambda i: (0, i)),
        pl.BlockSpec((1, 2 * D), lambda i: (0, 0)),
        pl.BlockSpec(memory_space=pltpu.SMEM),
    ],
    out_specs=[pl.BlockSpec((BC,), lambda i: (i,))],
    out_shape=[jax.ShapeDtypeStruct((N_ROWS,), jnp.float32)],
)


CW = 512
SPW = 31232
NSLAB = SPW // CW
NITER = (NSLAB - 1) // 2
RSC = SPW * NW
NTAIL = N_ROWS - RSC

_mesh = plsc.VectorSubcoreMesh(core_axis_name="c", subcore_axis_name="s")


@functools.partial(
    pl.kernel,
    mesh=_mesh,
    out_type=[
        jax.ShapeDtypeStruct((RSC,), jnp.float32),
        jax.ShapeDtypeStruct((NTAIL,), jnp.float32),
    ],
    compiler_params=pltpu.CompilerParams(
        needs_layout_passes=False, use_tc_tiling_on_sc=True
    ),
    scratch_types=[
        pltpu.VMEM((2, D, CW), jnp.float32),
        pltpu.VMEM((SPW,), jnp.float32),
        pltpu.VMEM((D,), jnp.float32),
        pltpu.VMEM((D, 512), jnp.float32),
        pltpu.VMEM((D, 64), jnp.float32),
        pltpu.VMEM((NTAIL,), jnp.float32),
        pltpu.SemaphoreType.DMA,
        pltpu.SemaphoreType.DMA,
        pltpu.SemaphoreType.DMA,
    ],
)
def _sc_matvec(itt_hbm, wi_hbm, yib_hbm, yit_hbm,
               slab, out_v, wi_v, tail_a, tail_b, tail_y, sem0, sem1, sem2):
    wid = lax.axis_index("s") * 2 + lax.axis_index("c")
    base = wid * SPW

    pltpu.sync_copy(wi_hbm, wi_v)
    wlo = wi_v[pl.ds(0, L)]
    whi = wi_v[pl.ds(L, L)]
    ws = [wlo[k] for k in range(L)] + [whi[k] for k in range(L)]

    sems = (sem0, sem1)

    def wait_and_compute(g, buf):
        col = base + g * CW
        pltpu.make_async_copy(
            itt_hbm.at[:, pl.ds(col, CW)], slab.at[buf], sems[buf]
        ).wait()
        sref = slab.at[buf]
        for c in range(CW // L):
            acc = jnp.zeros((L,), jnp.float32)
            for j in range(D):
                acc = acc + sref[j, pl.ds(c * L, L)] * ws[j]
            out_v[pl.ds(g * CW + c * L, L)] = acc
        return col

    for buf in range(2):
        pltpu.async_copy(
            itt_hbm.at[:, pl.ds(base + buf * CW, CW)], slab.at[buf], sems[buf]
        )

    @pl.loop(0, NITER)
    def _(it):
        for buf in range(2):
            g = it * 2 + buf
            col = wait_and_compute(g, buf)

            @pl.when(g + 2 < NSLAB)
            def _():
                pltpu.async_copy(
                    itt_hbm.at[:, pl.ds(col + 2 * CW, CW)], slab.at[buf], sems[buf]
                )

    wait_and_compute(NSLAB - 1, 0)

    pltpu.sync_copy(out_v, yib_hbm.at[pl.ds(base, SPW)])

    @pl.when(wid == 0)
    def _():
        pltpu.async_copy(itt_hbm.at[:, pl.ds(RSC, 512)], tail_a, sem2).wait()
        pltpu.async_copy(itt_hbm.at[:, pl.ds(RSC + 512, 64)], tail_b, sem2).wait()
        for c in range(512 // L):
            acc = jnp.zeros((L,), jnp.float32)
            for j in range(D):
                acc = acc + tail_a[j, pl.ds(c * L, L)] * ws[j]
            tail_y[pl.ds(c * L, L)] = acc
        for c in range(64 // L):
            acc = jnp.zeros((L,), jnp.float32)
            for j in range(D):
                acc = acc + tail_b[j, pl.ds(c * L, L)] * ws[j]
            tail_y[pl.ds(512 + c * L, L)] = acc
        pltpu.sync_copy(tail_y, yit_hbm)


BPW = B // NW
NIDX = 128
NCHUNK = BPW // NIDX


@functools.partial(
    pl.kernel,
    mesh=_mesh,
    out_type=jax.ShapeDtypeStruct((B,), jnp.float32),
    compiler_params=pltpu.CompilerParams(
        needs_layout_passes=False, use_tc_tiling_on_sc=False
    ),
    scratch_types=[
        pltpu.VMEM((NCHUNK, NIDX), jnp.int32),
        pltpu.VMEM((NCHUNK, NIDX), jnp.int32),
        pltpu.VMEM((NCHUNK, NIDX), jnp.int32),
        pltpu.VMEM((BPW,), jnp.float32),
        pltpu.VMEM((BPW,), jnp.float32),
        pltpu.VMEM((NTAIL,), jnp.float32),
        pltpu.VMEM((BPW,), jnp.float32),
        pltpu.SemaphoreType.DMA,
    ],
)
def _sc_gather_add(uid_hbm, iid_hbm, yu_hbm, yib_hbm, yit_hbm, out_hbm,
                   idx_u, idx_i, idx_b, vals_u, vals_i, yit_v, out_v, sem):
    wid = lax.axis_index("s") * 2 + lax.axis_index("c")
    base = wid * BPW

    pltpu.sync_copy(uid_hbm.at[wid], idx_u)
    pltpu.sync_copy(iid_hbm.at[wid], idx_i)
    pltpu.sync_copy(yit_hbm, yit_v)

    for k in range(NCHUNK):
        for c in range(NIDX // L):
            sl = pl.ds(c * L, L)
            idx_b[k, sl] = jnp.minimum(idx_i[k, sl], RSC - 1)

    copies = []
    for k in range(NCHUNK):
        dst = pl.ds(k * NIDX, NIDX)
        copies.append(pltpu.async_copy(yu_hbm.at[idx_u.at[k]], vals_u.at[dst], sem))
        copies.append(pltpu.async_copy(yib_hbm.at[idx_b.at[k]], vals_i.at[dst], sem))
    for cp in copies:
        cp.wait()

    zero = jnp.zeros((L,), jnp.int32)
    for g in range(BPW // L):
        sl = pl.ds(g * L, L)
        iidv = idx_i[g // (NIDX // L), pl.ds((g % (NIDX // L)) * L, L)]
        idx_t = jnp.maximum(iidv - RSC, zero)
        vt = plsc.load_gather(yit_v, [idx_t])
        yi = jnp.where(iidv < RSC, vals_i[sl], vt)
        out_v[sl] = vals_u[sl] + yi

    pltpu.sync_copy(out_v, out_hbm.at[pl.ds(base, BPW)])


def kernel(user_ids, item_ids, user_table, item_table, W, b):
    utt = user_table.T
    itt = item_table.T
    bs = b.reshape(1).astype(jnp.float32)
    (yu,) = _matvec(utt, W.astype(jnp.float32), bs)
    yib, yit = _sc_matvec(itt, W[0, D:].astype(jnp.float32))

    uid = user_ids.astype(jnp.int32).reshape(NW, NCHUNK, NIDX)
    iid = item_ids.astype(jnp.int32).reshape(NW, NCHUNK, NIDX)
    return _sc_gather_add(uid, iid, yu, yib, yit)


# speedup vs baseline: 1.5269x; 1.5269x over previous
import functools

import jax
import jax.numpy as jnp
from jax import lax
from jax.experimental import pallas as pl
from jax.experimental.pallas import tpu as pltpu
from jax.experimental.pallas import tpu_sc as plsc

N_ROWS = 1000000
D = 32
B = 16384
L = 16
NW = 32


BC = 100352
GRID = (N_ROWS + BC - 1) // BC


def _matvec_body(utt_ref, w_ref, b_ref, yu_ref):
    u = utt_ref[...]
    wu = w_ref[0:1, :D]
    yu = jax.lax.dot_general(wu, u, (((1,), (0,)), ((), ())),
                             preferred_element_type=jnp.float32)
    yu_ref[...] = yu[0] + b_ref[0]


_matvec = pl.pallas_call(
    _matvec_body,
    grid=(GRID,),
    in_specs=[
        pl.BlockSpec((D, BC), lambda i: (0, i)),
        pl.BlockSpec((1, 2 * D), lambda i: (0, 0)),
        pl.BlockSpec(memory_space=pltpu.SMEM),
    ],
    out_specs=[pl.BlockSpec((BC,), lambda i: (i,))],
    out_shape=[jax.ShapeDtypeStruct((N_ROWS,), jnp.float32)],
)

BC2 = 77824


def _matvec_item_body(itt_ref, w_ref, yi_ref):
    it = itt_ref[...]
    wi = w_ref[0:1, D:]
    yi = jax.lax.dot_general(wi, it, (((1,), (0,)), ((), ())),
                             preferred_element_type=jnp.float32)
    yi_ref[...] = yi[0]


def _make_matvec_item(rtc):
    return pl.pallas_call(
        _matvec_item_body,
        grid=(rtc // BC2,),
        in_specs=[
            pl.BlockSpec((D, BC2), lambda i: (0, i)),
            pl.BlockSpec((1, 2 * D), lambda i: (0, 0)),
        ],
        out_specs=[pl.BlockSpec((BC2,), lambda i: (i,))],
        out_shape=[jax.ShapeDtypeStruct((rtc,), jnp.float32)],
    )


CW = 512
NSLAB = 23
SPW = NSLAB * CW
NITER = (NSLAB - 1) // 2
RTC = 999424 - SPW * NW
RSC = SPW * NW
NTAIL = N_ROWS - RTC - RSC

_mesh = plsc.VectorSubcoreMesh(core_axis_name="c", subcore_axis_name="s")


@functools.partial(
    pl.kernel,
    mesh=_mesh,
    out_type=[
        jax.ShapeDtypeStruct((RSC,), jnp.float32),
        jax.ShapeDtypeStruct((NTAIL,), jnp.float32),
    ],
    compiler_params=pltpu.CompilerParams(
        needs_layout_passes=False, use_tc_tiling_on_sc=True
    ),
    scratch_types=[
        pltpu.VMEM((2, D, CW), jnp.float32),
        pltpu.VMEM((SPW,), jnp.float32),
        pltpu.VMEM((D,), jnp.float32),
        pltpu.VMEM((D, 512), jnp.float32),
        pltpu.VMEM((D, 64), jnp.float32),
        pltpu.VMEM((NTAIL,), jnp.float32),
        pltpu.SemaphoreType.DMA,
        pltpu.SemaphoreType.DMA,
        pltpu.SemaphoreType.DMA,
    ],
)
def _sc_matvec(itt_hbm, wi_hbm, yib_hbm, yit_hbm,
               slab, out_v, wi_v, tail_a, tail_b, tail_y, sem0, sem1, sem2):
    wid = lax.axis_index("s") * 2 + lax.axis_index("c")
    base = RTC + wid * SPW

    pltpu.sync_copy(wi_hbm, wi_v)
    wlo = wi_v[pl.ds(0, L)]
    whi = wi_v[pl.ds(L, L)]
    ws = [wlo[k] for k in range(L)] + [whi[k] for k in range(L)]

    sems = (sem0, sem1)

    def wait_and_compute(g, buf):
        col = base + g * CW
        pltpu.make_async_copy(
            itt_hbm.at[:, pl.ds(col, CW)], slab.at[buf], sems[buf]
        ).wait()
        sref = slab.at[buf]
        for c in range(CW // L):
            acc = jnp.zeros((L,), jnp.float32)
            for j in range(D):
                acc = acc + sref[j, pl.ds(c * L, L)] * ws[j]
            out_v[pl.ds(g * CW + c * L, L)] = acc
        return col

    for buf in range(2):
        pltpu.async_copy(
            itt_hbm.at[:, pl.ds(base + buf * CW, CW)], slab.at[buf], sems[buf]
        )

    @pl.loop(0, NITER)
    def _(it):
        for buf in range(2):
            g = it * 2 + buf
            col = wait_and_compute(g, buf)

            @pl.when(g + 2 < NSLAB)
            def _():
                pltpu.async_copy(
                    itt_hbm.at[:, pl.ds(col + 2 * CW, CW)], slab.at[buf], sems[buf]
                )

    wait_and_compute(NSLAB - 1, 0)

    pltpu.sync_copy(out_v, yib_hbm.at[pl.ds(wid * SPW, SPW)])

    @pl.when(wid == 0)
    def _():
        t0 = RTC + RSC
        pltpu.async_copy(itt_hbm.at[:, pl.ds(t0, 512)], tail_a, sem2).wait()
        pltpu.async_copy(itt_hbm.at[:, pl.ds(t0 + 512, 64)], tail_b, sem2).wait()
        for c in range(512 // L):
            acc = jnp.zeros((L,), jnp.float32)
            for j in range(D):
                acc = acc + tail_a[j, pl.ds(c * L, L)] * ws[j]
            tail_y[pl.ds(c * L, L)] = acc
        for c in range(64 // L):
            acc = jnp.zeros((L,), jnp.float32)
            for j in range(D):
                acc = acc + tail_b[j, pl.ds(c * L, L)] * ws[j]
            tail_y[pl.ds(512 + c * L, L)] = acc
        pltpu.sync_copy(tail_y, yit_hbm)


BPW = B // NW
NIDX = 128
NCHUNK = BPW // NIDX


@functools.partial(
    pl.kernel,
    mesh=_mesh,
    out_type=jax.ShapeDtypeStruct((B,), jnp.float32),
    compiler_params=pltpu.CompilerParams(
        needs_layout_passes=False, use_tc_tiling_on_sc=False
    ),
    scratch_types=[
        pltpu.VMEM((NCHUNK, NIDX), jnp.int32),
        pltpu.VMEM((NCHUNK, NIDX), jnp.int32),
        pltpu.VMEM((NCHUNK, NIDX), jnp.int32),
        pltpu.VMEM((NCHUNK, NIDX), jnp.int32),
        pltpu.VMEM((BPW,), jnp.float32),
        pltpu.VMEM((BPW,), jnp.float32),
        pltpu.VMEM((BPW,), jnp.float32),
        pltpu.VMEM((NTAIL,), jnp.float32),
        pltpu.VMEM((BPW,), jnp.float32),
        pltpu.SemaphoreType.DMA,
    ],
)
def _sc_gather_add(uid_hbm, iid_hbm, yu_hbm, yia_hbm, yib_hbm, yit_hbm, out_hbm,
                   idx_u, idx_i, idx_a, idx_b, vals_u, vals_a, vals_b, yit_v,
                   out_v, sem):
    wid = lax.axis_index("s") * 2 + lax.axis_index("c")
    base = wid * BPW

    pltpu.sync_copy(uid_hbm.at[wid], idx_u)
    pltpu.sync_copy(iid_hbm.at[wid], idx_i)
    pltpu.sync_copy(yit_hbm, yit_v)

    zero = jnp.zeros((L,), jnp.int32)
    for k in range(NCHUNK):
        for c in range(NIDX // L):
            sl = pl.ds(c * L, L)
            iidv = idx_i[k, sl]
            idx_a[k, sl] = jnp.minimum(iidv, RTC - 1)
            idx_b[k, sl] = jnp.minimum(jnp.maximum(iidv - RTC, zero), RSC - 1)

    copies = []
    for k in range(NCHUNK):
        dst = pl.ds(k * NIDX, NIDX)
        copies.append(pltpu.async_copy(yu_hbm.at[idx_u.at[k]], vals_u.at[dst], sem))
        copies.append(pltpu.async_copy(yia_hbm.at[idx_a.at[k]], vals_a.at[dst], sem))
        copies.append(pltpu.async_copy(yib_hbm.at[idx_b.at[k]], vals_b.at[dst], sem))
    for cp in copies:
        cp.wait()

    for g in range(BPW // L):
        sl = pl.ds(g * L, L)
        iidv = idx_i[g // (NIDX // L), pl.ds((g % (NIDX // L)) * L, L)]
        idx_t = jnp.maximum(iidv - (RTC + RSC), zero)
        vt = plsc.load_gather(yit_v, [idx_t])
        yi = jnp.where(iidv < RTC, vals_a[sl],
                       jnp.where(iidv < RTC + RSC, vals_b[sl], vt))
        out_v[sl] = vals_u[sl] + yi

    pltpu.sync_copy(out_v, out_hbm.at[pl.ds(base, BPW)])


def kernel(user_ids, item_ids, user_table, item_table, W, b):
    utt = user_table.T
    itt = item_table.T
    bs = b.reshape(1).astype(jnp.float32)
    wf = W.astype(jnp.float32)
    (yu,) = _matvec(utt, wf, bs)
    (yia,) = _make_matvec_item(RTC)(itt, wf)
    yib, yit = _sc_matvec(itt, W[0, D:].astype(jnp.float32))

    uid = user_ids.astype(jnp.int32).reshape(NW, NCHUNK, NIDX)
    iid = item_ids.astype(jnp.int32).reshape(NW, NCHUNK, NIDX)
    return _sc_gather_add(uid, iid, yu, yia, yib, yit)


# speedup vs baseline: 2.5972x; 1.7009x over previous
import functools

import jax
import jax.numpy as jnp
from jax import lax
from jax.experimental import pallas as pl
from jax.experimental.pallas import tpu as pltpu
from jax.experimental.pallas import tpu_sc as plsc

N_ROWS = 1000000
D = 32
B = 16384


BC = 100352
GRID = (N_ROWS + BC - 1) // BC


def _matvec_body(utt_ref, itt_ref, w_ref, b_ref, yu_ref, yi_ref):
    u = utt_ref[...]
    it = itt_ref[...]
    wu = w_ref[0:1, :D]
    wi = w_ref[0:1, D:]
    yu = jax.lax.dot_general(wu, u, (((1,), (0,)), ((), ())),
                             preferred_element_type=jnp.float32)
    yi = jax.lax.dot_general(wi, it, (((1,), (0,)), ((), ())),
                             preferred_element_type=jnp.float32)
    yu_ref[...] = yu[0] + b_ref[0]
    yi_ref[...] = yi[0]


_matvec = pl.pallas_call(
    _matvec_body,
    grid=(GRID,),
    in_specs=[
        pl.BlockSpec((D, BC), lambda i: (0, i)),
        pl.BlockSpec((D, BC), lambda i: (0, i)),
        pl.BlockSpec((1, 2 * D), lambda i: (0, 0)),
        pl.BlockSpec(memory_space=pltpu.SMEM),
    ],
    out_specs=[
        pl.BlockSpec((BC,), lambda i: (i,)),
        pl.BlockSpec((BC,), lambda i: (i,)),
    ],
    out_shape=[
        jax.ShapeDtypeStruct((N_ROWS,), jnp.float32),
        jax.ShapeDtypeStruct((N_ROWS,), jnp.float32),
    ],
)


L = 16
NW = 32
BPW = B // NW
NIDX = 128
NCHUNK = BPW // NIDX

_mesh = plsc.VectorSubcoreMesh(core_axis_name="c", subcore_axis_name="s")


@functools.partial(
    pl.kernel,
    mesh=_mesh,
    out_type=jax.ShapeDtypeStruct((B,), jnp.float32),
    compiler_params=pltpu.CompilerParams(
        needs_layout_passes=False, use_tc_tiling_on_sc=False
    ),
    scratch_types=[
        pltpu.VMEM((NCHUNK, NIDX), jnp.int32),
        pltpu.VMEM((NCHUNK, NIDX), jnp.int32),
        pltpu.VMEM((BPW,), jnp.float32),
        pltpu.VMEM((BPW,), jnp.float32),
        pltpu.VMEM((BPW,), jnp.float32),
        pltpu.SemaphoreType.DMA,
    ],
)
def _sc_gather_add(uid_hbm, iid_hbm, yu_hbm, yi_hbm, out_hbm,
                   idx_u, idx_i, vals_u, vals_i, out_v, sem):
    wid = lax.axis_index("s") * 2 + lax.axis_index("c")
    base = wid * BPW

    pltpu.sync_copy(uid_hbm.at[wid], idx_u)
    pltpu.sync_copy(iid_hbm.at[wid], idx_i)

    copies = []
    for k in range(NCHUNK):
        dst = pl.ds(k * NIDX, NIDX)
        copies.append(pltpu.async_copy(yu_hbm.at[idx_u.at[k]], vals_u.at[dst], sem))
        copies.append(pltpu.async_copy(yi_hbm.at[idx_i.at[k]], vals_i.at[dst], sem))
    for cp in copies:
        cp.wait()

    for g in range(BPW // L):
        sl = pl.ds(g * L, L)
        out_v[sl] = vals_u[sl] + vals_i[sl]

    pltpu.sync_copy(out_v, out_hbm.at[pl.ds(base, BPW)])


def kernel(user_ids, item_ids, user_table, item_table, W, b):
    utt = user_table.T
    itt = item_table.T
    bs = b.reshape(1).astype(jnp.float32)
    yu, yi = _matvec(utt, itt, W.astype(jnp.float32), bs)

    uid = user_ids.astype(jnp.int32).reshape(NW, NCHUNK, NIDX)
    iid = item_ids.astype(jnp.int32).reshape(NW, NCHUNK, NIDX)
    return _sc_gather_add(uid, iid, yu, yi)
